# baseline (speedup 1.0000x reference)
import functools

import jax
import jax.numpy as jnp
import numpy as np
from jax import lax
from jax.experimental import pallas as pl
from jax.experimental.pallas import tpu as pltpu
from jax.experimental.pallas import tpu_sc as plsc

B = 1024
V = 100000
D = 64
NT = 4
P = 8
L = 8
OUT = 300

NCORES = 2
NSUB = 16
NLANE = 16
NW = NCORES * NSUB
ITEMS = 2 * B
IPW = ITEMS // NW
GRP = 1
NGRP = IPW // GRP
CH = 4
CHROWS = (GRP * NT * P * L) // CH


def _dot_partial(node, rows_ref, row):
    acc = node[0] * rows_ref[row, pl.ds(0, NLANE)]
    for c in range(1, D // NLANE):
        acc = acc + node[c] * rows_ref[row, pl.ds(c * NLANE, NLANE)]
    return acc


_GDNUMS = lax.GatherDimensionNumbers(
    offset_dims=(), collapsed_slice_dims=(0,), start_index_map=(0,))


def _lane_perm(x, k):
    idx = jnp.bitwise_xor(lax.iota(jnp.int32, NLANE), k)
    return lax.gather(x, idx[:, None], _GDNUMS, (1,),
                      mode=lax.GatherScatterMode.PROMISE_IN_BOUNDS)


def _hsum(x):
    for k in (1, 2, 4, 8):
        x = x + _lane_perm(x, k)
    return x


def _hmax(x):
    for k in (1, 2, 4, 8):
        x = jnp.maximum(x, _lane_perm(x, k))
    return x


def _hmin_i32(x):
    for k in (1, 2, 4, 8):
        x = jnp.minimum(x, _lane_perm(x, k))
    return x


def _sc_body(table, idx_feat, idx_node, scal, out,
             idx_vf, idx_vn, rows_f, rows_n, scal_v, out_v,
             sems):
    wid = lax.axis_index("s") * NCORES + lax.axis_index("c")
    base = wid * IPW

    pltpu.sync_copy(scal, scal_v)
    pltpu.sync_copy(idx_node.at[pl.ds(base, IPW)], idx_vn)
    pltpu.sync_copy(idx_feat.at[pl.ds(wid * NGRP, NGRP)], idx_vf)
    pltpu.async_copy(table.at[idx_vn], rows_n, sems.at[0]).wait()

    def fetch(slot, grp):
        for k in range(CH):
            pltpu.async_copy(table.at[idx_vf.at[grp, k]],
                             rows_f.at[slot, k], sems.at[slot])

    def drain(slot, grp):
        for k in range(CH):
            pltpu.make_async_copy(table.at[idx_vf.at[grp, k]],
                                  rows_f.at[slot, k], sems.at[slot]).wait()

    def compute(slot, grp, j):
        g = grp * GRP + j
        node = [rows_n[g, pl.ds(c * NLANE, NLANE)]
                for c in range(D // NLANE)]
        lane = lax.iota(jnp.int32, NLANE)
        vvec = scal_v[pl.ds(0, NLANE)]

        ctx = [jnp.zeros((NLANE,), jnp.float32) for _ in range(D // NLANE)]
        for t in range(NT):
            fr0 = j * (NT * P * L) + t * (P * L)
            rt = rows_f.at[slot, fr0 // CHROWS]
            off = fr0 % CHROWS

            def pbody(p, sv):
                acc = _dot_partial(node, rt, off + p * L)
                for l in range(1, L):
                    acc = acc + _dot_partial(node, rt, off + p * L + l)
                return jnp.where(lane == p, _hsum(acc), sv)

            sv = lax.fori_loop(0, P, pbody,
                               jnp.full((NLANE,), -3e38, jnp.float32))
            m = _hmax(sv)
            cand = jnp.where((sv == m) & (lane < P), lane,
                             jnp.int32(NLANE))
            rb = off + _hmin_i32(cand)[0] * L

            x = jnp.zeros((NLANE,), jnp.float32)
            for l in range(L):
                x = jnp.where(lane == l,
                              _hsum(_dot_partial(node, rt, rb + l)), x)
            xm = x + jnp.where(x != 0.0, 0.0, -9999.0).astype(jnp.float32)
            xm = jnp.where(lane < L, xm, jnp.float32(-3e38))
            e = jnp.exp(xm - _hmax(xm))
            e = jnp.where(lane < L, e, 0.0).astype(jnp.float32)
            w = e / _hsum(e)
            wt = vvec[L + t]
            coeff = w * vvec * wt

            for l in range(L):
                cl = coeff[l]
                for c in range(D // NLANE):
                    ctx[c] = ctx[c] + cl * rt[rb + l, pl.ds(c * NLANE, NLANE)]

        for c in range(D // NLANE):
            out_v[g, pl.ds(c * NLANE, NLANE)] = node[c]
            out_v[g, pl.ds(D + c * NLANE, NLANE)] = ctx[c]

    fetch(0, 0)

    def body(gg, _):
        grp0 = gg * 2
        fetch(1, grp0 + 1)
        drain(0, grp0)
        for jj in range(GRP):
            compute(0, grp0, jj)

        @pl.when(grp0 + 2 < NGRP)
        def _():
            fetch(0, grp0 + 2)

        drain(1, grp0 + 1)
        for jj in range(GRP):
            compute(1, grp0 + 1, jj)
        return 0

    lax.fori_loop(0, NGRP // 2, body, 0)
    pltpu.sync_copy(out_v, out.at[pl.ds(base, IPW)])


@functools.partial(jax.jit, static_argnames=())
def _sc_contextual(table, idx_feat, idx_node, scal):
    mesh = plsc.VectorSubcoreMesh(core_axis_name="c", subcore_axis_name="s")
    kern = pl.kernel(
        _sc_body,
        out_type=jax.ShapeDtypeStruct((ITEMS, 2 * D), jnp.float32),
        mesh=mesh,
        scratch_types=[
            pltpu.VMEM((NGRP, CH, CHROWS), jnp.int32),
            pltpu.VMEM((IPW,), jnp.int32),
            pltpu.VMEM((2, CH, CHROWS, D), jnp.float32),
            pltpu.VMEM((IPW, D), jnp.float32),
            pltpu.VMEM((NLANE,), jnp.float32),
            pltpu.VMEM((IPW, 2 * D), jnp.float32),
            pltpu.SemaphoreType.DMA((2,)),
        ],
        compiler_params=pltpu.CompilerParams(use_tc_tiling_on_sc=False),
    )
    return kern(table, idx_feat, idx_node, scal)


def _repack_body(t_ref, o_ref):
    o_ref[...] = jnp.concatenate([t_ref[::2, :], t_ref[1::2, :]], axis=1)


@jax.jit
def _repack(table):
    return pl.pallas_call(
        _repack_body,
        grid=(50,),
        in_specs=[pl.BlockSpec((V // 50, D), lambda i: (i, 0))],
        out_specs=pl.BlockSpec((V // 100, 2 * D), lambda i: (i, 0)),
        out_shape=jax.ShapeDtypeStruct((V // 2, 2 * D), jnp.float32),
    )(table)


def _tc_body(c_ref, w_ref, b_ref, o_ref):
    c0 = c_ref[:B, :]
    c1 = c_ref[B:, :]
    w = w_ref[...]
    b = b_ref[...]
    r0 = lax.dot_general(c0, w, (((1,), (1,)), ((), ())),
                         preferred_element_type=jnp.float32,
                         precision=lax.Precision.HIGHEST) + b[None, :]
    r1 = lax.dot_general(c1, w, (((1,), (1,)), ((), ())),
                         preferred_element_type=jnp.float32,
                         precision=lax.Precision.HIGHEST) + b[None, :]
    eps = 1e-8
    num = jnp.sum(r0 * r1, axis=1)
    n0 = jnp.maximum(jnp.sqrt(jnp.sum(r0 * r0, axis=1)), eps)
    n1 = jnp.maximum(jnp.sqrt(jnp.sum(r1 * r1, axis=1)), eps)
    o_ref[...] = num / (n0 * n1)


@jax.jit
def _tc_cosine(ctx, w_out, b_out):
    return pl.pallas_call(
        _tc_body,
        out_shape=jax.ShapeDtypeStruct((B,), jnp.float32),
    )(ctx, w_out, b_out)


def kernel(nodes, features, emb_table, W_out, b_out, v,
           w_rootpath, w_children, w_obj_neighbours):
    idx_feat = (features.astype(jnp.int32)
                .transpose(1, 0, 2, 3, 4)
                .reshape(ITEMS // GRP, CH, CHROWS))
    idx_node = nodes.astype(jnp.int32).transpose(1, 0).reshape(ITEMS)
    wd = 1.0 - w_rootpath - w_children - w_obj_neighbours
    scal = jnp.concatenate([
        v.astype(jnp.float32),
        w_rootpath.astype(jnp.float32), w_children.astype(jnp.float32),
        w_obj_neighbours.astype(jnp.float32), wd.astype(jnp.float32),
        jnp.zeros((4,), jnp.float32),
    ])
    tbl = _repack(emb_table.astype(jnp.float32)).reshape(V, D)
    ctx = _sc_contextual(tbl, idx_feat, idx_node, scal)
    return _tc_cosine(ctx, W_out.astype(jnp.float32),
                      b_out.astype(jnp.float32))


# speedup vs baseline: 1.1195x; 1.1195x over previous
import functools

import jax
import jax.numpy as jnp
import numpy as np
from jax import lax
from jax.experimental import pallas as pl
from jax.experimental.pallas import tpu as pltpu
from jax.experimental.pallas import tpu_sc as plsc

B = 1024
V = 100000
D = 64
NT = 4
P = 8
L = 8
OUT = 300

NCORES = 2
NSUB = 16
NLANE = 16
NW = NCORES * NSUB
ITEMS = 2 * B
IPW = ITEMS // NW
GRP = 1
NGRP = IPW // GRP
CH = 4
CHROWS = (GRP * NT * P * L) // CH


def _dot_partial(node, rows_ref, row):
    acc = node[0] * rows_ref[row, pl.ds(0, NLANE)]
    for c in range(1, D // NLANE):
        acc = acc + node[c] * rows_ref[row, pl.ds(c * NLANE, NLANE)]
    return acc


_GDNUMS = lax.GatherDimensionNumbers(
    offset_dims=(), collapsed_slice_dims=(0,), start_index_map=(0,))


def _lane_perm(x, k):
    idx = jnp.bitwise_xor(lax.iota(jnp.int32, NLANE), k)
    return lax.gather(x, idx[:, None], _GDNUMS, (1,),
                      mode=lax.GatherScatterMode.PROMISE_IN_BOUNDS)


def _hsum(x):
    for k in (1, 2, 4, 8):
        x = x + _lane_perm(x, k)
    return x


def _hmax(x):
    for k in (1, 2, 4, 8):
        x = jnp.maximum(x, _lane_perm(x, k))
    return x


def _hmin_i32(x):
    for k in (1, 2, 4, 8):
        x = jnp.minimum(x, _lane_perm(x, k))
    return x


def _sc_body(table, idx_feat, idx_node, scal, out,
             idx_vf, idx_vn, rows_f, rows_n, scal_v, out_v,
             sems):
    wid = lax.axis_index("s") * NCORES + lax.axis_index("c")
    base = wid * IPW

    pltpu.sync_copy(scal, scal_v)
    pltpu.sync_copy(idx_node.at[pl.ds(base, IPW)], idx_vn)
    pltpu.sync_copy(idx_feat.at[pl.ds(wid * NGRP, NGRP)], idx_vf)
    pltpu.async_copy(table.at[idx_vn], rows_n, sems.at[0, 0]).wait()

    def fetch(slot, grp):
        for k in range(CH):
            pltpu.async_copy(table.at[idx_vf.at[grp, k]],
                             rows_f.at[slot, k], sems.at[slot, k])

    def drain_chunk(slot, grp, k):
        pltpu.make_async_copy(table.at[idx_vf.at[grp, k]],
                              rows_f.at[slot, k], sems.at[slot, k]).wait()

    def compute(slot, grp, j):
        g = grp * GRP + j
        node = [rows_n[g, pl.ds(c * NLANE, NLANE)]
                for c in range(D // NLANE)]
        lane = lax.iota(jnp.int32, NLANE)
        vvec = scal_v[pl.ds(0, NLANE)]

        ctx = [jnp.zeros((NLANE,), jnp.float32) for _ in range(D // NLANE)]
        prev_chunk = -1
        for t in range(NT):
            fr0 = j * (NT * P * L) + t * (P * L)
            if fr0 // CHROWS != prev_chunk:
                prev_chunk = fr0 // CHROWS
                drain_chunk(slot, grp, prev_chunk)
            rt = rows_f.at[slot, fr0 // CHROWS]
            off = fr0 % CHROWS

            def pbody(p, sv):
                acc = _dot_partial(node, rt, off + p * L)
                for l in range(1, L):
                    acc = acc + _dot_partial(node, rt, off + p * L + l)
                return jnp.where(lane == p, _hsum(acc), sv)

            sv = lax.fori_loop(0, P, pbody,
                               jnp.full((NLANE,), -3e38, jnp.float32))
            m = _hmax(sv)
            cand = jnp.where((sv == m) & (lane < P), lane,
                             jnp.int32(NLANE))
            rb = off + _hmin_i32(cand)[0] * L

            x = jnp.zeros((NLANE,), jnp.float32)
            for l in range(L):
                x = jnp.where(lane == l,
                              _hsum(_dot_partial(node, rt, rb + l)), x)
            xm = x + jnp.where(x != 0.0, 0.0, -9999.0).astype(jnp.float32)
            xm = jnp.where(lane < L, xm, jnp.float32(-3e38))
            e = jnp.exp(xm - _hmax(xm))
            e = jnp.where(lane < L, e, 0.0).astype(jnp.float32)
            w = e / _hsum(e)
            wt = vvec[L + t]
            coeff = w * vvec * wt

            for l in range(L):
                cl = coeff[l]
                for c in range(D // NLANE):
                    ctx[c] = ctx[c] + cl * rt[rb + l, pl.ds(c * NLANE, NLANE)]

        for c in range(D // NLANE):
            out_v[g, pl.ds(c * NLANE, NLANE)] = node[c]
            out_v[g, pl.ds(D + c * NLANE, NLANE)] = ctx[c]

    fetch(0, 0)

    def body(gg, _):
        grp0 = gg * 2
        fetch(1, grp0 + 1)
        for jj in range(GRP):
            compute(0, grp0, jj)

        @pl.when(grp0 + 2 < NGRP)
        def _():
            fetch(0, grp0 + 2)

        for jj in range(GRP):
            compute(1, grp0 + 1, jj)
        return 0

    lax.fori_loop(0, NGRP // 2, body, 0)
    pltpu.sync_copy(out_v, out.at[pl.ds(base, IPW)])


@functools.partial(jax.jit, static_argnames=())
def _sc_contextual(table, idx_feat, idx_node, scal):
    mesh = plsc.VectorSubcoreMesh(core_axis_name="c", subcore_axis_name="s")
    kern = pl.kernel(
        _sc_body,
        out_type=jax.ShapeDtypeStruct((ITEMS, 2 * D), jnp.float32),
        mesh=mesh,
        scratch_types=[
            pltpu.VMEM((NGRP, CH, CHROWS), jnp.int32),
            pltpu.VMEM((IPW,), jnp.int32),
            pltpu.VMEM((2, CH, CHROWS, D), jnp.float32),
            pltpu.VMEM((IPW, D), jnp.float32),
            pltpu.VMEM((NLANE,), jnp.float32),
            pltpu.VMEM((IPW, 2 * D), jnp.float32),
            pltpu.SemaphoreType.DMA((2, CH)),
        ],
        compiler_params=pltpu.CompilerParams(use_tc_tiling_on_sc=False),
    )
    return kern(table, idx_feat, idx_node, scal)


def _tc_body(c_ref, w_ref, b_ref, o_ref):
    c0 = c_ref[:B, :]
    c1 = c_ref[B:, :]
    w = w_ref[...]
    b = b_ref[...]
    r0 = lax.dot_general(c0, w, (((1,), (1,)), ((), ())),
                         preferred_element_type=jnp.float32,
                         precision=lax.Precision.HIGHEST) + b[None, :]
    r1 = lax.dot_general(c1, w, (((1,), (1,)), ((), ())),
                         preferred_element_type=jnp.float32,
                         precision=lax.Precision.HIGHEST) + b[None, :]
    eps = 1e-8
    num = jnp.sum(r0 * r1, axis=1)
    n0 = jnp.maximum(jnp.sqrt(jnp.sum(r0 * r0, axis=1)), eps)
    n1 = jnp.maximum(jnp.sqrt(jnp.sum(r1 * r1, axis=1)), eps)
    o_ref[...] = num / (n0 * n1)


@jax.jit
def _tc_cosine(ctx, w_out, b_out):
    return pl.pallas_call(
        _tc_body,
        out_shape=jax.ShapeDtypeStruct((B,), jnp.float32),
    )(ctx, w_out, b_out)


def kernel(nodes, features, emb_table, W_out, b_out, v,
           w_rootpath, w_children, w_obj_neighbours):
    idx_feat = (features.astype(jnp.int32)
                .transpose(1, 0, 2, 3, 4)
                .reshape(ITEMS // GRP, CH, CHROWS))
    idx_node = nodes.astype(jnp.int32).transpose(1, 0).reshape(ITEMS)
    wd = 1.0 - w_rootpath - w_children - w_obj_neighbours
    scal = jnp.concatenate([
        v.astype(jnp.float32),
        w_rootpath.astype(jnp.float32), w_children.astype(jnp.float32),
        w_obj_neighbours.astype(jnp.float32), wd.astype(jnp.float32),
        jnp.zeros((4,), jnp.float32),
    ])
    ctx = _sc_contextual(emb_table.astype(jnp.float32),
                         idx_feat, idx_node, scal)
    return _tc_cosine(ctx, W_out.astype(jnp.float32),
                      b_out.astype(jnp.float32))


# speedup vs baseline: 1.1363x; 1.0150x over previous
import functools

import jax
import jax.numpy as jnp
import numpy as np
from jax import lax
from jax.experimental import pallas as pl
from jax.experimental.pallas import tpu as pltpu
from jax.experimental.pallas import tpu_sc as plsc

B = 1024
V = 100000
D = 64
NT = 4
P = 8
L = 8
OUT = 300

NCORES = 2
NSUB = 16
NLANE = 16
NW = NCORES * NSUB
ITEMS = 2 * B
IPW = ITEMS // NW
GRP = 1
NGRP = IPW // GRP
CH = 4
CHROWS = (GRP * NT * P * L) // CH


def _dot_partial(node, rows_ref, row):
    acc = node[0] * rows_ref[row, pl.ds(0, NLANE)]
    for c in range(1, D // NLANE):
        acc = acc + node[c] * rows_ref[row, pl.ds(c * NLANE, NLANE)]
    return acc


_GDNUMS = lax.GatherDimensionNumbers(
    offset_dims=(), collapsed_slice_dims=(0,), start_index_map=(0,))


def _lane_perm(x, k):
    idx = jnp.bitwise_xor(lax.iota(jnp.int32, NLANE), k)
    return lax.gather(x, idx[:, None], _GDNUMS, (1,),
                      mode=lax.GatherScatterMode.PROMISE_IN_BOUNDS)


def _hsum(x):
    for k in (1, 2, 4, 8):
        x = x + _lane_perm(x, k)
    return x


def _hmax(x):
    for k in (1, 2, 4, 8):
        x = jnp.maximum(x, _lane_perm(x, k))
    return x


def _hmin_i32(x):
    for k in (1, 2, 4, 8):
        x = jnp.minimum(x, _lane_perm(x, k))
    return x


def _sc_body(table, idx_feat, idx_node, scal, out,
             idx_vf, idx_vn, rows_f, rows_n, scal_v, out_v,
             sems):
    wid = lax.axis_index("s") * NCORES + lax.axis_index("c")
    base = wid * IPW

    pltpu.sync_copy(scal, scal_v)
    pltpu.sync_copy(idx_node.at[pl.ds(base, IPW)], idx_vn)
    pltpu.sync_copy(idx_feat.at[pl.ds(wid * NGRP, NGRP)], idx_vf)
    pltpu.async_copy(table.at[idx_vn], rows_n, sems.at[0]).wait()

    def fetch(slot, grp):
        for k in range(CH):
            pltpu.async_copy(table.at[idx_vf.at[grp, k]],
                             rows_f.at[slot, k], sems.at[slot])

    def drain(slot, grp):
        for k in range(CH):
            pltpu.make_async_copy(table.at[idx_vf.at[grp, k]],
                                  rows_f.at[slot, k], sems.at[slot]).wait()

    def compute(slot, grp, j):
        g = grp * GRP + j
        node = [rows_n[g, pl.ds(c * NLANE, NLANE)]
                for c in range(D // NLANE)]
        lane = lax.iota(jnp.int32, NLANE)
        vvec = scal_v[pl.ds(0, NLANE)]

        ctx = [jnp.zeros((NLANE,), jnp.float32) for _ in range(D // NLANE)]
        for t in range(NT):
            fr0 = j * (NT * P * L) + t * (P * L)
            rt = rows_f.at[slot, fr0 // CHROWS]
            off = fr0 % CHROWS

            def pbody(p, sv):
                acc = _dot_partial(node, rt, off + p * L)
                for l in range(1, L):
                    acc = acc + _dot_partial(node, rt, off + p * L + l)
                return jnp.where(lane == p, _hsum(acc), sv)

            sv = lax.fori_loop(0, P, pbody,
                               jnp.full((NLANE,), -3e38, jnp.float32))
            m = _hmax(sv)
            cand = jnp.where((sv == m) & (lane < P), lane,
                             jnp.int32(NLANE))
            rb = off + _hmin_i32(cand)[0] * L

            x = jnp.zeros((NLANE,), jnp.float32)
            for l in range(L):
                x = jnp.where(lane == l,
                              _hsum(_dot_partial(node, rt, rb + l)), x)
            xm = x + jnp.where(x != 0.0, 0.0, -9999.0).astype(jnp.float32)
            xm = jnp.where(lane < L, xm, jnp.float32(-3e38))
            e = jnp.exp(xm - _hmax(xm))
            e = jnp.where(lane < L, e, 0.0).astype(jnp.float32)
            w = e / _hsum(e)
            wt = vvec[L + t]
            coeff = w * vvec * wt

            for l in range(L):
                cl = coeff[l]
                for c in range(D // NLANE):
                    ctx[c] = ctx[c] + cl * rt[rb + l, pl.ds(c * NLANE, NLANE)]

        for c in range(D // NLANE):
            out_v[g, pl.ds(c * NLANE, NLANE)] = node[c]
            out_v[g, pl.ds(D + c * NLANE, NLANE)] = ctx[c]

    fetch(0, 0)

    def body(gg, _):
        grp0 = gg * 2
        fetch(1, grp0 + 1)
        drain(0, grp0)
        for jj in range(GRP):
            compute(0, grp0, jj)

        @pl.when(grp0 + 2 < NGRP)
        def _():
            fetch(0, grp0 + 2)

        drain(1, grp0 + 1)
        for jj in range(GRP):
            compute(1, grp0 + 1, jj)
        return 0

    lax.fori_loop(0, NGRP // 2, body, 0)
    pltpu.sync_copy(out_v, out.at[pl.ds(base, IPW)])


@functools.partial(jax.jit, static_argnames=())
def _sc_contextual(table, idx_feat, idx_node, scal):
    mesh = plsc.VectorSubcoreMesh(core_axis_name="c", subcore_axis_name="s")
    kern = pl.kernel(
        _sc_body,
        out_type=jax.ShapeDtypeStruct((ITEMS, 2 * D), jnp.float32),
        mesh=mesh,
        scratch_types=[
            pltpu.VMEM((NGRP, CH, CHROWS), jnp.int32),
            pltpu.VMEM((IPW,), jnp.int32),
            pltpu.VMEM((2, CH, CHROWS, D), jnp.float32),
            pltpu.VMEM((IPW, D), jnp.float32),
            pltpu.VMEM((NLANE,), jnp.float32),
            pltpu.VMEM((IPW, 2 * D), jnp.float32),
            pltpu.SemaphoreType.DMA((2,)),
        ],
        compiler_params=pltpu.CompilerParams(use_tc_tiling_on_sc=False),
    )
    return kern(table, idx_feat, idx_node, scal)


def _tc_body(c_ref, w_ref, b_ref, o_ref):
    c0 = c_ref[:B, :]
    c1 = c_ref[B:, :]
    w = w_ref[...]
    b = b_ref[...]
    r0 = lax.dot_general(c0, w, (((1,), (1,)), ((), ())),
                         preferred_element_type=jnp.float32,
                         precision=lax.Precision.HIGHEST) + b[None, :]
    r1 = lax.dot_general(c1, w, (((1,), (1,)), ((), ())),
                         preferred_element_type=jnp.float32,
                         precision=lax.Precision.HIGHEST) + b[None, :]
    eps = 1e-8
    num = jnp.sum(r0 * r1, axis=1)
    n0 = jnp.maximum(jnp.sqrt(jnp.sum(r0 * r0, axis=1)), eps)
    n1 = jnp.maximum(jnp.sqrt(jnp.sum(r1 * r1, axis=1)), eps)
    o_ref[...] = num / (n0 * n1)


@jax.jit
def _tc_cosine(ctx, w_out, b_out):
    return pl.pallas_call(
        _tc_body,
        out_shape=jax.ShapeDtypeStruct((B,), jnp.float32),
    )(ctx, w_out, b_out)


def kernel(nodes, features, emb_table, W_out, b_out, v,
           w_rootpath, w_children, w_obj_neighbours):
    idx_feat = (features.astype(jnp.int32)
                .transpose(1, 0, 2, 3, 4)
                .reshape(ITEMS // GRP, CH, CHROWS))
    idx_node = nodes.astype(jnp.int32).transpose(1, 0).reshape(ITEMS)
    wd = 1.0 - w_rootpath - w_children - w_obj_neighbours
    scal = jnp.concatenate([
        v.astype(jnp.float32),
        w_rootpath.astype(jnp.float32), w_children.astype(jnp.float32),
        w_obj_neighbours.astype(jnp.float32), wd.astype(jnp.float32),
        jnp.zeros((4,), jnp.float32),
    ])
    ctx = _sc_contextual(emb_table.astype(jnp.float32),
                         idx_feat, idx_node, scal)
    return _tc_cosine(ctx, W_out.astype(jnp.float32),
                      b_out.astype(jnp.float32))
